# baseline (device time: 18514 ns/iter reference)
import numpy as np

import jax
import jax.numpy as jnp
from jax import lax
from jax.experimental import pallas as pl
from jax.experimental.pallas import tpu as pltpu

N_DEV = 16
B, H, D = 8, 8, 64
PAGES_PER_DEV = 64
BLOCK = 16
T_LOCAL = PAGES_PER_DEV * BLOCK
TH = T_LOCAL * H
BH = B * H
NCHUNK = 4
CR = BH // NCHUNK
LANES = 128

_cc = np.arange(TH)
_CONST = np.concatenate(
    [
        (np.arange(PAGES_PER_DEV)[:, None] == (_cc[None, :] >> 7)),
        (np.arange(H)[:, None] == (_cc[None, :] & 7)),
    ],
    axis=0,
).astype(np.float32)


def kernel(Q, K, V, bt, lens):
    hbm = pltpu.MemorySpace.HBM
    k2 = pltpu.with_memory_space_constraint(K.reshape(TH, D), hbm)
    v2 = pltpu.with_memory_space_constraint(V.reshape(TH, D), hbm)
    q64 = pltpu.with_memory_space_constraint(Q.reshape(BH, D), hbm)
    cmat = pltpu.with_memory_space_constraint(
        jnp.asarray(_CONST, dtype=jnp.bfloat16), hbm)
    btc = pltpu.with_memory_space_constraint(bt, hbm)
    lens1 = pltpu.with_memory_space_constraint(
        lens.reshape(1, B), hbm)

    def body(q_ref, k_ref, v_ref, c_ref, bt_ref, lens_ref, out_ref,
             comm_ref, kv_vmem, c_vmem, q_vmem, bt_vmem, lens_vmem, res_vmem,
             dma_sems, send_sems, recv_sems):
        my = lax.axis_index("i")
        scope = jax.named_scope

        with scope("ph_dma_start"):
            btcopy = pltpu.make_async_copy(bt_ref, bt_vmem, dma_sems.at[0])
            lcopy = pltpu.make_async_copy(lens_ref, lens_vmem, dma_sems.at[5])
            ccopy = pltpu.make_async_copy(c_ref, c_vmem, dma_sems.at[1])
            qcopy = pltpu.make_async_copy(q_ref, q_vmem, dma_sems.at[2])
            kcopy = pltpu.make_async_copy(k_ref, kv_vmem.at[0], dma_sems.at[3])
            vcopy = pltpu.make_async_copy(v_ref, kv_vmem.at[1], dma_sems.at[4])
            btcopy.start()
            lcopy.start()
            ccopy.start()
            qcopy.start()
            kcopy.start()
            vcopy.start()

        barrier = pltpu.get_barrier_semaphore()
        for o in range(1, N_DEV):
            peer = lax.rem(my + o, N_DEV)
            pl.semaphore_signal(
                barrier, inc=1, device_id=(peer,),
                device_id_type=pl.DeviceIdType.MESH,
            )

        with scope("ph_btdma_wait"):
            btcopy.wait()
            lcopy.wait()
        with scope("ph_wbuild"):
            base_f = lax.convert_element_type(my * PAGES_PER_DEV, jnp.float32)
            btf_t = bt_vmem[:, :].astype(jnp.float32).T
            lens_f = lens_vmem[:, :].astype(jnp.float32)
            jt = lax.broadcasted_iota(
                jnp.int32, (PAGES_PER_DEV, B), 0).astype(jnp.float32)
            btm_t = jnp.where(jt < lens_f, btf_t, -1.0)
            pidf = base_f + lax.broadcasted_iota(
                jnp.int32, (PAGES_PER_DEV, PAGES_PER_DEV), 1
            ).astype(jnp.float32)
            rows = []
            for i in range(B):
                cmp = (btm_t[:, i:i + 1] == pidf).astype(jnp.float32)
                rows.append(jnp.sum(cmp, axis=0, keepdims=True))
            w_page = jnp.concatenate(rows, axis=0)

        with scope("ph_wexpand"):
            erep = (lax.broadcasted_iota(jnp.int32, (BH, B), 0) // H
                    == lax.broadcasted_iota(jnp.int32, (BH, B), 1)
                    ).astype(jnp.float32)
            wp64 = lax.dot_general(
                erep, w_page, (((1,), (0,)), ((), ())),
                preferred_element_type=jnp.float32,
            )
            ccopy.wait()
            wpb = lax.dot_general(
                wp64.astype(jnp.bfloat16), c_vmem[0:PAGES_PER_DEV, :],
                (((1,), (0,)), ((), ())),
                preferred_element_type=jnp.float32,
            )
            ahead = (lax.broadcasted_iota(jnp.int32, (BH, H), 0) % H
                     == lax.broadcasted_iota(jnp.int32, (BH, H), 1)
                     ).astype(jnp.bfloat16)
            hm = lax.dot_general(
                ahead, c_vmem[PAGES_PER_DEV:PAGES_PER_DEV + H, :],
                (((1,), (0,)), ((), ())),
                preferred_element_type=jnp.float32,
            )
            wm_big = wpb * hm

        with scope("ph_kdma_wait"):
            qcopy.wait()
            kcopy.wait()
            vcopy.wait()
        with scope("ph_barrier"):
            pl.semaphore_wait(barrier, N_DEV - 1)

        scale = jnp.float32(D ** -0.5)
        rdmas = {}
        for c in range(NCHUNK):
            rs = c * CR
            with scope(f"ph_chunk{c}_compute"):
                qf = q_vmem[rs:rs + CR, :] * scale
                s_c = lax.dot_general(
                    qf, kv_vmem[0], (((1,), (1,)), ((), ())),
                    preferred_element_type=jnp.float32,
                )
                m_c = jnp.max(s_c, axis=1, keepdims=True)
                p_c = jnp.exp(s_c - m_c) * wm_big[rs:rs + CR, :]
                l_c = jnp.sum(p_c, axis=1, keepdims=True)
                o_c = lax.dot_general(
                    p_c, kv_vmem[1], (((1,), (0,)), ((), ())),
                    preferred_element_type=jnp.float32,
                )
                comm_ref[c, 0, :, 0:D] = o_c.astype(jnp.bfloat16)
                comm_ref[c, 0, :, D:D + 1] = m_c.astype(jnp.bfloat16)
                comm_ref[c, 0, :, D + 1:D + 2] = l_c.astype(jnp.bfloat16)
            with scope(f"ph_chunk{c}_issue"):
                for o in range(1, N_DEV):
                    target = lax.rem(my + o, N_DEV)
                    slot = N_DEV - o
                    rdmas[c, o] = pltpu.make_async_remote_copy(
                        src_ref=comm_ref.at[c, 0],
                        dst_ref=comm_ref.at[c, slot],
                        send_sem=send_sems.at[c, o],
                        recv_sem=recv_sems.at[c, slot],
                        device_id=(target,),
                        device_id_type=pl.DeviceIdType.MESH,
                    )
                    rdmas[c, o].start()

        def lse_partial(c, lo, hi):
            o_a = comm_ref[c, lo:hi, :, 0:D].astype(jnp.float32)
            m_a = comm_ref[c, lo:hi, :, D:D + 1].astype(jnp.float32)
            l_a = comm_ref[c, lo:hi, :, D + 1:D + 2].astype(jnp.float32)
            m_p = jnp.max(m_a, axis=0)
            sc = jnp.exp(m_a - m_p[None])
            return (jnp.sum(o_a * sc, axis=0), jnp.sum(l_a * sc, axis=0), m_p)

        for c in range(NCHUNK):
            with scope(f"ph_wait{c}a"):
                for o in range(1, 9):
                    rdmas[c, o].wait()
            with scope(f"ph_merge{c}a"):
                o1, l1, m1 = lse_partial(c, 8, N_DEV)
            with scope(f"ph_wait{c}b"):
                for o in range(9, N_DEV):
                    rdmas[c, o].wait()
            with scope(f"ph_merge{c}b"):
                o2, l2, m2 = lse_partial(c, 0, 8)
                m_g = jnp.maximum(m1, m2)
                s1 = jnp.exp(m1 - m_g)
                s2 = jnp.exp(m2 - m_g)
                res_c = (o1 * s1 + o2 * s2) / (l1 * s1 + l2 * s2)
                for i in range(CR // H):
                    res_vmem[2 * c + i, 0, :, :] = res_c[i * H:(i + 1) * H, :]

        with scope("ph_store"):
            outcopy = pltpu.make_async_copy(res_vmem, out_ref, dma_sems.at[6])
            outcopy.start()
            outcopy.wait()

    return pl.pallas_call(
        body,
        out_shape=jax.ShapeDtypeStruct((B, 1, H, D), jnp.float32),
        in_specs=[
            pl.BlockSpec(memory_space=pltpu.MemorySpace.HBM),
            pl.BlockSpec(memory_space=pltpu.MemorySpace.HBM),
            pl.BlockSpec(memory_space=pltpu.MemorySpace.HBM),
            pl.BlockSpec(memory_space=pltpu.MemorySpace.HBM),
            pl.BlockSpec(memory_space=pltpu.MemorySpace.HBM),
            pl.BlockSpec(memory_space=pltpu.MemorySpace.HBM),
        ],
        out_specs=pl.BlockSpec(memory_space=pltpu.MemorySpace.HBM),
        scratch_shapes=[
            pltpu.VMEM((NCHUNK, N_DEV, CR, LANES), jnp.bfloat16),
            pltpu.VMEM((2, TH, D), jnp.float32),
            pltpu.VMEM((BH + H, TH), jnp.bfloat16),
            pltpu.VMEM((BH, D), jnp.float32),
            pltpu.VMEM((B, PAGES_PER_DEV), jnp.int32),
            pltpu.VMEM((1, B), jnp.int32),
            pltpu.VMEM((B, 1, H, D), jnp.float32),
            pltpu.SemaphoreType.DMA((7,)),
            pltpu.SemaphoreType.DMA((NCHUNK, N_DEV)),
            pltpu.SemaphoreType.DMA((NCHUNK, N_DEV)),
        ],
        compiler_params=pltpu.CompilerParams(collective_id=0),
    )(q64, k2, v2, cmat, btc, lens1)


# device time: 15978 ns/iter; 1.1587x vs baseline; 1.1587x over previous
import numpy as np

import jax
import jax.numpy as jnp
from jax import lax
from jax.experimental import pallas as pl
from jax.experimental.pallas import tpu as pltpu

N_DEV = 16
B, H, D = 8, 8, 64
PAGES_PER_DEV = 64
BLOCK = 16
T_LOCAL = PAGES_PER_DEV * BLOCK
TH = T_LOCAL * H
BH = B * H
NCHUNK = 2
CR = BH // NCHUNK
LANES = 128

_cc = np.arange(TH)
_CONST = np.concatenate(
    [
        (np.arange(PAGES_PER_DEV)[:, None] == (_cc[None, :] >> 7)),
        (np.arange(H)[:, None] == (_cc[None, :] & 7)),
    ],
    axis=0,
).astype(np.float32)


def kernel(Q, K, V, bt, lens):
    hbm = pltpu.MemorySpace.HBM
    k2 = pltpu.with_memory_space_constraint(K.reshape(TH, D), hbm)
    v2 = pltpu.with_memory_space_constraint(V.reshape(TH, D), hbm)
    q64 = pltpu.with_memory_space_constraint(Q.reshape(BH, D), hbm)
    cmat = pltpu.with_memory_space_constraint(
        jnp.asarray(_CONST, dtype=jnp.bfloat16), hbm)
    btc = pltpu.with_memory_space_constraint(bt, hbm)
    lens1 = pltpu.with_memory_space_constraint(
        lens.reshape(1, B), hbm)

    def body(q_ref, k_ref, v_ref, c_ref, bt_ref, lens_ref, out_ref,
             comm_ref, kv_vmem, c_vmem, q_vmem, bt_vmem, lens_vmem, res_vmem,
             dma_sems, send_sems, recv_sems):
        my = lax.axis_index("i")
        scope = jax.named_scope

        with scope("ph_dma_start"):
            btcopy = pltpu.make_async_copy(bt_ref, bt_vmem, dma_sems.at[0])
            lcopy = pltpu.make_async_copy(lens_ref, lens_vmem, dma_sems.at[5])
            ccopy = pltpu.make_async_copy(c_ref, c_vmem, dma_sems.at[1])
            qcopy = pltpu.make_async_copy(q_ref, q_vmem, dma_sems.at[2])
            kcopy = pltpu.make_async_copy(k_ref, kv_vmem.at[0], dma_sems.at[3])
            vcopy = pltpu.make_async_copy(v_ref, kv_vmem.at[1], dma_sems.at[4])
            btcopy.start()
            lcopy.start()
            ccopy.start()
            qcopy.start()
            kcopy.start()
            vcopy.start()

        barrier = pltpu.get_barrier_semaphore()
        for o in range(1, N_DEV):
            peer = lax.rem(my + o, N_DEV)
            pl.semaphore_signal(
                barrier, inc=1, device_id=(peer,),
                device_id_type=pl.DeviceIdType.MESH,
            )

        with scope("ph_btdma_wait"):
            btcopy.wait()
            lcopy.wait()
        with scope("ph_wbuild"):
            base_f = lax.convert_element_type(my * PAGES_PER_DEV, jnp.float32)
            btf_t = bt_vmem[:, :].astype(jnp.float32).T
            lens_f = lens_vmem[:, :].astype(jnp.float32)
            jt = lax.broadcasted_iota(
                jnp.int32, (PAGES_PER_DEV, B), 0).astype(jnp.float32)
            btm_t = jnp.where(jt < lens_f, btf_t, -1.0)
            pidf = base_f + lax.broadcasted_iota(
                jnp.int32, (PAGES_PER_DEV, PAGES_PER_DEV), 1
            ).astype(jnp.float32)
            rows = []
            for i in range(B):
                cmp = (btm_t[:, i:i + 1] == pidf).astype(jnp.float32)
                rows.append(jnp.sum(cmp, axis=0, keepdims=True))
            w_page = jnp.concatenate(rows, axis=0)

        with scope("ph_wexpand"):
            erep = (lax.broadcasted_iota(jnp.int32, (BH, B), 0) // H
                    == lax.broadcasted_iota(jnp.int32, (BH, B), 1)
                    ).astype(jnp.float32)
            wp64 = lax.dot_general(
                erep, w_page, (((1,), (0,)), ((), ())),
                preferred_element_type=jnp.float32,
            )
            ccopy.wait()
            wpb = lax.dot_general(
                wp64.astype(jnp.bfloat16), c_vmem[0:PAGES_PER_DEV, :],
                (((1,), (0,)), ((), ())),
                preferred_element_type=jnp.float32,
            )
            ahead = (lax.broadcasted_iota(jnp.int32, (BH, H), 0) % H
                     == lax.broadcasted_iota(jnp.int32, (BH, H), 1)
                     ).astype(jnp.bfloat16)
            hm = lax.dot_general(
                ahead, c_vmem[PAGES_PER_DEV:PAGES_PER_DEV + H, :],
                (((1,), (0,)), ((), ())),
                preferred_element_type=jnp.float32,
            )
            wm_big = wpb * hm

        with scope("ph_kdma_wait"):
            qcopy.wait()
            kcopy.wait()
            vcopy.wait()
        with scope("ph_barrier"):
            pl.semaphore_wait(barrier, N_DEV - 1)

        scale = jnp.float32(D ** -0.5)
        rdmas = {}
        for c in range(NCHUNK):
            rs = c * CR
            with scope(f"ph_chunk{c}_compute"):
                qf = q_vmem[rs:rs + CR, :] * scale
                s_c = lax.dot_general(
                    qf, kv_vmem[0], (((1,), (1,)), ((), ())),
                    preferred_element_type=jnp.float32,
                )
                m_c = jnp.max(s_c, axis=1, keepdims=True)
                p_c = jnp.exp(s_c - m_c) * wm_big[rs:rs + CR, :]
                l_c = jnp.sum(p_c, axis=1, keepdims=True)
                o_c = lax.dot_general(
                    p_c, kv_vmem[1], (((1,), (0,)), ((), ())),
                    preferred_element_type=jnp.float32,
                )
                comm_ref[c, 0, :, 0:D] = o_c.astype(jnp.bfloat16)
                comm_ref[c, 0, :, D:D + 1] = m_c.astype(jnp.bfloat16)
                comm_ref[c, 0, :, D + 1:D + 2] = l_c.astype(jnp.bfloat16)
            with scope(f"ph_chunk{c}_issue"):
                for o in range(1, N_DEV):
                    target = lax.rem(my + o, N_DEV)
                    slot = N_DEV - o
                    rdmas[c, o] = pltpu.make_async_remote_copy(
                        src_ref=comm_ref.at[c, 0],
                        dst_ref=comm_ref.at[c, slot],
                        send_sem=send_sems.at[c, o],
                        recv_sem=recv_sems.at[c, slot],
                        device_id=(target,),
                        device_id_type=pl.DeviceIdType.MESH,
                    )
                    rdmas[c, o].start()

        def lse_partial(c, lo, hi):
            o_a = comm_ref[c, lo:hi, :, 0:D].astype(jnp.float32)
            m_a = comm_ref[c, lo:hi, :, D:D + 1].astype(jnp.float32)
            l_a = comm_ref[c, lo:hi, :, D + 1:D + 2].astype(jnp.float32)
            m_p = jnp.max(m_a, axis=0)
            sc = jnp.exp(m_a - m_p[None])
            return (jnp.sum(o_a * sc, axis=0), jnp.sum(l_a * sc, axis=0), m_p)

        for c in range(NCHUNK):
            with scope(f"ph_wait{c}a"):
                for o in range(1, 9):
                    rdmas[c, o].wait()
            with scope(f"ph_merge{c}a"):
                o1, l1, m1 = lse_partial(c, 8, N_DEV)
            with scope(f"ph_wait{c}b"):
                for o in range(9, N_DEV):
                    rdmas[c, o].wait()
            with scope(f"ph_merge{c}b"):
                o2, l2, m2 = lse_partial(c, 0, 8)
                m_g = jnp.maximum(m1, m2)
                s1 = jnp.exp(m1 - m_g)
                s2 = jnp.exp(m2 - m_g)
                res_c = (o1 * s1 + o2 * s2) / (l1 * s1 + l2 * s2)
                for i in range(CR // H):
                    res_vmem[2 * c + i, 0, :, :] = res_c[i * H:(i + 1) * H, :]

        with scope("ph_store"):
            outcopy = pltpu.make_async_copy(res_vmem, out_ref, dma_sems.at[6])
            outcopy.start()
            outcopy.wait()

    return pl.pallas_call(
        body,
        out_shape=jax.ShapeDtypeStruct((B, 1, H, D), jnp.float32),
        in_specs=[
            pl.BlockSpec(memory_space=pltpu.MemorySpace.HBM),
            pl.BlockSpec(memory_space=pltpu.MemorySpace.HBM),
            pl.BlockSpec(memory_space=pltpu.MemorySpace.HBM),
            pl.BlockSpec(memory_space=pltpu.MemorySpace.HBM),
            pl.BlockSpec(memory_space=pltpu.MemorySpace.HBM),
            pl.BlockSpec(memory_space=pltpu.MemorySpace.HBM),
        ],
        out_specs=pl.BlockSpec(memory_space=pltpu.MemorySpace.HBM),
        scratch_shapes=[
            pltpu.VMEM((NCHUNK, N_DEV, CR, LANES), jnp.bfloat16),
            pltpu.VMEM((2, TH, D), jnp.float32),
            pltpu.VMEM((BH + H, TH), jnp.bfloat16),
            pltpu.VMEM((BH, D), jnp.float32),
            pltpu.VMEM((B, PAGES_PER_DEV), jnp.int32),
            pltpu.VMEM((1, B), jnp.int32),
            pltpu.VMEM((B, 1, H, D), jnp.float32),
            pltpu.SemaphoreType.DMA((7,)),
            pltpu.SemaphoreType.DMA((NCHUNK, N_DEV)),
            pltpu.SemaphoreType.DMA((NCHUNK, N_DEV)),
        ],
        compiler_params=pltpu.CompilerParams(collective_id=0),
    )(q64, k2, v2, cmat, btc, lens1)
